# Initial kernel scaffold; baseline (speedup 1.0000x reference)
import jax
import jax.numpy as jnp
from jax.experimental import pallas as pl


def kernel(pos, heading, valid, agent_shape, token_traj):
    raise NotImplementedError("write your pallas kernel here")


# speedup vs baseline: 4.5270x; 4.5270x over previous
import jax
import jax.numpy as jnp
from jax import lax
from jax.experimental import pallas as pl

N_STEP = 89
SHIFT = 8
A_BLK = 256


def _body(px_r, py_r, hd_r, vf_r, ash_r, tok_r,
          vm_o, idx_o, gpx_o, gpy_o, gh_o):
    T = tok_r.shape[1]
    n_out = vm_o.shape[1]
    iota_t = lax.broadcasted_iota(jnp.int32, (A_BLK, T), 1)

    l = ash_r[:, 0:1] / 2.0
    w = ash_r[:, 1:2] / 2.0
    corners = ((l, w), (l, -w), (-l, -w), (-l, w))

    pp_x = px_r[:, 0:1]
    pp_y = py_r[:, 0:1]
    ph = hd_r[:, 0:1]

    for j in range(n_out):
        si = j + 1
        vmask = vf_r[:, si - 1:si] * vf_r[:, si:si + 1]
        vb = vmask > 0.0

        h_i = hd_r[:, si:si + 1]
        c_i = jnp.cos(h_i)
        s_i = jnp.sin(h_i)
        px_i = px_r[:, si:si + 1]
        py_i = py_r[:, si:si + 1]

        cx = [x * c_i - y * s_i + px_i for (x, y) in corners]
        cy = [x * s_i + y * c_i + py_i for (x, y) in corners]

        pc = jnp.cos(ph)
        ps = jnp.sin(ph)

        d = None
        for k in range(4):
            tx = tok_r[k:k + 1, :]
            ty = tok_r[4 + k:5 + k, :]
            gx = tx * pc - ty * ps + pp_x
            gy = tx * ps + ty * pc + pp_y
            dx = gx - cx[k]
            dy = gy - cy[k]
            dk = jnp.sqrt(dx * dx + dy * dy)
            d = dk if d is None else d + dk

        m = jnp.min(d, axis=1, keepdims=True)
        idx = jnp.min(jnp.where(d == m, iota_t, T), axis=1, keepdims=True)
        onehot = (iota_t == idx).astype(jnp.float32)

        sx = []
        sy = []
        for k in range(4):
            tx = tok_r[k:k + 1, :]
            ty = tok_r[4 + k:5 + k, :]
            sx.append(jnp.sum(onehot * tx, axis=1, keepdims=True))
            sy.append(jnp.sum(onehot * ty, axis=1, keepdims=True))
        wx = [sx[k] * pc - sy[k] * ps + pp_x for k in range(4)]
        wy = [sx[k] * ps + sy[k] * pc + pp_y for k in range(4)]

        dxx = wx[0] - wx[3]
        dyy = wy[0] - wy[3]
        nh = jnp.arctan2(dyy, dxx)
        mean_x = (wx[0] + wx[1] + wx[2] + wx[3]) / 4.0
        mean_y = (wy[0] + wy[1] + wy[2] + wy[3]) / 4.0

        ph = jnp.where(vb, nh, h_i)
        pp_x = jnp.where(vb, mean_x, px_i)
        pp_y = jnp.where(vb, mean_y, py_i)

        vm_o[:, j:j + 1] = vmask
        idx_o[:, j:j + 1] = idx
        gpx_o[:, j:j + 1] = jnp.where(vb, pp_x, 0.0)
        gpy_o[:, j:j + 1] = jnp.where(vb, pp_y, 0.0)
        gh_o[:, j:j + 1] = jnp.where(vb, ph, 0.0)


def kernel(pos, heading, valid, agent_shape, token_traj):
    A = pos.shape[0]
    T = token_traj.shape[0]
    ns = (N_STEP + SHIFT - 1) // SHIFT
    n_out = ns - 1

    px = pos[:, ::SHIFT, 0]
    py = pos[:, ::SHIFT, 1]
    hd = heading[:, ::SHIFT]
    vf = valid[:, ::SHIFT].astype(jnp.float32)
    tok8 = jnp.concatenate(
        [token_traj[:, :, 0].T, token_traj[:, :, 1].T], axis=0)

    grid = (A // A_BLK,)
    ab = lambda a: (a, 0)
    outs = pl.pallas_call(
        _body,
        grid=grid,
        in_specs=[
            pl.BlockSpec((A_BLK, ns), ab),
            pl.BlockSpec((A_BLK, ns), ab),
            pl.BlockSpec((A_BLK, ns), ab),
            pl.BlockSpec((A_BLK, ns), ab),
            pl.BlockSpec((A_BLK, 2), ab),
            pl.BlockSpec((8, T), lambda a: (0, 0)),
        ],
        out_specs=[
            pl.BlockSpec((A_BLK, n_out), ab),
            pl.BlockSpec((A_BLK, n_out), ab),
            pl.BlockSpec((A_BLK, n_out), ab),
            pl.BlockSpec((A_BLK, n_out), ab),
            pl.BlockSpec((A_BLK, n_out), ab),
        ],
        out_shape=[
            jax.ShapeDtypeStruct((A, n_out), jnp.float32),
            jax.ShapeDtypeStruct((A, n_out), jnp.int32),
            jax.ShapeDtypeStruct((A, n_out), jnp.float32),
            jax.ShapeDtypeStruct((A, n_out), jnp.float32),
            jax.ShapeDtypeStruct((A, n_out), jnp.float32),
        ],
    )(px, py, hd, vf, agent_shape, tok8)

    vm, idx, gpx, gpy, gh = outs
    valid_mask = vm.T > 0.0
    gt_idx = idx.T
    gt_pos = jnp.stack([gpx.T, gpy.T], axis=-1)
    gt_head = gh.T
    return valid_mask, gt_idx, gt_pos, gt_head
